# baseline (speedup 1.0000x reference)
import functools

import jax
import jax.numpy as jnp
from jax import lax
from jax.experimental import pallas as pl
from jax.experimental.pallas import tpu as pltpu
from jax.experimental.pallas import tpu_sc as plsc

DIM = 64
CHUNK = 800
NROW = 2
NIDX = 4


@functools.cache
def _make_gather(n_total: int, dim: int):
    info = plsc.get_sparse_core_info()
    nw = info.num_cores * info.num_subcores
    per_w = n_total // nw
    n_chunks = per_w // CHUNK
    assert per_w * nw == n_total and n_chunks * CHUNK == per_w
    assert n_chunks % NIDX == 0 and n_chunks // NIDX >= 2

    mesh = plsc.VectorSubcoreMesh(core_axis_name="c", subcore_axis_name="s")

    @functools.partial(
        pl.kernel,
        mesh=mesh,
        out_type=jax.ShapeDtypeStruct((n_total, dim), jnp.float32),
        scratch_types=[
            pltpu.VMEM((NIDX, CHUNK), jnp.int32),
            pltpu.VMEM((NROW, CHUNK, dim), jnp.float32),
            pltpu.SemaphoreType.DMA,
            pltpu.SemaphoreType.DMA,
            pltpu.SemaphoreType.DMA,
            pltpu.SemaphoreType.DMA,
            pltpu.SemaphoreType.DMA,
            pltpu.SemaphoreType.DMA,
            pltpu.SemaphoreType.DMA,
            pltpu.SemaphoreType.DMA,
        ],
        compiler_params=pltpu.CompilerParams(use_tc_tiling_on_sc=False),
    )
    def gather_kernel(idx_hbm, table_hbm, out_hbm, idx_v, rows_v,
                      isem0, isem1, isem2, isem3, gsem0, gsem1, osem0, osem1):
        isems = (isem0, isem1, isem2, isem3)
        gsems = (gsem0, gsem1)
        osems = (osem0, osem1)
        wid = lax.axis_index("s") * info.num_cores + lax.axis_index("c")
        base = wid * per_w

        def idx_load_start(c, j):
            pltpu.async_copy(
                idx_hbm.at[pl.ds(base + c * CHUNK, CHUNK)], idx_v.at[j], isems[j])

        def idx_load_wait(j):
            pltpu.make_async_copy(
                idx_hbm.at[pl.ds(base, CHUNK)], idx_v.at[j], isems[j]).wait()

        def out_write_wait(b):
            pltpu.make_async_copy(
                rows_v.at[b], out_hbm.at[pl.ds(base, CHUNK)], osems[b]).wait()

        def visit(c, k, skip_osem=False, prefetch=True):
            b = k % NROW
            j = k % NIDX
            if not skip_osem:
                out_write_wait(b)
            if prefetch:
                idx_load_start(c + NROW, (k + NROW) % NIDX)
            idx_load_wait(j)
            pltpu.async_copy(table_hbm.at[idx_v.at[j]], rows_v.at[b], gsems[b]).wait()
            pltpu.async_copy(
                rows_v.at[b], out_hbm.at[pl.ds(base + c * CHUNK, CHUNK)], osems[b])

        n_quads = n_chunks // NIDX

        idx_load_start(0, 0)
        idx_load_start(1, 1)
        for k in range(NIDX):
            visit(k, k, skip_osem=(k < NROW))

        def quad(o, carry):
            cb = o * NIDX
            for k in range(NIDX):
                visit(cb + k, k)
            return carry

        lax.fori_loop(1, n_quads - 1, quad, 0)

        cb = (n_quads - 1) * NIDX
        for k in range(NIDX):
            visit(cb + k, k, prefetch=(k < NIDX - NROW))

        out_write_wait(0)
        out_write_wait(1)

    return gather_kernel


def kernel(x, table):
    b, l = x.shape
    n = b * l
    flat = x.reshape(n).astype(jnp.int32)
    out = _make_gather(n, DIM)(flat, table)
    return out.reshape(b, l, DIM)


# speedup vs baseline: 1.6531x; 1.6531x over previous
import functools

import jax
import jax.numpy as jnp
from jax import lax
from jax.experimental import pallas as pl
from jax.experimental.pallas import tpu as pltpu
from jax.experimental.pallas import tpu_sc as plsc

DIM = 64
CHUNK = 800
NROW = 2
NIDX = 4


@functools.cache
def _make_gather(n_total: int, dim: int):
    info = plsc.get_sparse_core_info()
    nw = info.num_cores * info.num_subcores
    per_w = n_total // nw
    n_chunks = per_w // CHUNK
    assert per_w * nw == n_total and n_chunks * CHUNK == per_w
    assert n_chunks % NIDX == 0 and n_chunks // NIDX >= 2

    mesh = plsc.VectorSubcoreMesh(core_axis_name="c", subcore_axis_name="s")

    @functools.partial(
        pl.kernel,
        mesh=mesh,
        out_type=jax.ShapeDtypeStruct((n_total, 2 * dim), jnp.float32),
        scratch_types=[
            pltpu.VMEM((NIDX, CHUNK), jnp.int32),
            pltpu.VMEM((NROW, CHUNK, dim), jnp.float32),
            pltpu.SemaphoreType.DMA,
            pltpu.SemaphoreType.DMA,
            pltpu.SemaphoreType.DMA,
            pltpu.SemaphoreType.DMA,
            pltpu.SemaphoreType.DMA,
            pltpu.SemaphoreType.DMA,
            pltpu.SemaphoreType.DMA,
            pltpu.SemaphoreType.DMA,
        ],
        compiler_params=pltpu.CompilerParams(use_tc_tiling_on_sc=False),
    )
    def gather_kernel(idx_hbm, table_hbm, out_hbm, idx_v, rows_v,
                      isem0, isem1, isem2, isem3, gsem0, gsem1, osem0, osem1):
        isems = (isem0, isem1, isem2, isem3)
        gsems = (gsem0, gsem1)
        osems = (osem0, osem1)
        wid = lax.axis_index("s") * info.num_cores + lax.axis_index("c")
        base = wid * per_w

        def idx_load_start(c, j):
            pltpu.async_copy(
                idx_hbm.at[pl.ds(base + c * CHUNK, CHUNK)], idx_v.at[j], isems[j])

        def idx_load_wait(j):
            pltpu.make_async_copy(
                idx_hbm.at[pl.ds(base, CHUNK)], idx_v.at[j], isems[j]).wait()

        def out_write_wait(b):
            pltpu.make_async_copy(
                rows_v.at[b],
                out_hbm.at[pl.ds(base, CHUNK), pl.ds(0, dim)], osems[b]).wait()

        def visit(c, k, skip_osem=False, prefetch=True):
            b = k % NROW
            j = k % NIDX
            if not skip_osem:
                out_write_wait(b)
            if prefetch:
                idx_load_start(c + NROW, (k + NROW) % NIDX)
            idx_load_wait(j)
            pltpu.async_copy(
                table_hbm.at[idx_v.at[j]], rows_v.at[b], gsems[b]).wait()
            pltpu.async_copy(
                rows_v.at[b],
                out_hbm.at[pl.ds(base + c * CHUNK, CHUNK), pl.ds(0, dim)],
                osems[b])

        n_quads = n_chunks // NIDX

        idx_load_start(0, 0)
        idx_load_start(1, 1)
        for k in range(NIDX):
            visit(k, k, skip_osem=(k < NROW))

        def quad(o, carry):
            cb = o * NIDX
            for k in range(NIDX):
                visit(cb + k, k)
            return carry

        lax.fori_loop(1, n_quads - 1, quad, 0)

        cb = (n_quads - 1) * NIDX
        for k in range(NIDX):
            visit(cb + k, k, prefetch=(k < NIDX - NROW))

        out_write_wait(0)
        out_write_wait(1)

    return gather_kernel


def kernel(x, table):
    b, l = x.shape
    n = b * l
    flat = x.reshape(n).astype(jnp.int32)
    out_pad = _make_gather(n, DIM)(flat, table)
    return out_pad[:, :DIM].reshape(b, l, DIM)


# speedup vs baseline: 1.6566x; 1.0021x over previous
import functools

import jax
import jax.numpy as jnp
from jax import lax
from jax.experimental import pallas as pl
from jax.experimental.pallas import tpu as pltpu
from jax.experimental.pallas import tpu_sc as plsc

DIM = 64
CHUNK = 400
NROW = 4
NIDX = 4


@functools.cache
def _make_gather(n_total: int, dim: int):
    info = plsc.get_sparse_core_info()
    nw = info.num_cores * info.num_subcores
    per_w = n_total // nw
    n_chunks = per_w // CHUNK
    assert per_w * nw == n_total and n_chunks * CHUNK == per_w
    assert n_chunks % NIDX == 0 and n_chunks // NIDX >= 2

    mesh = plsc.VectorSubcoreMesh(core_axis_name="c", subcore_axis_name="s")

    @functools.partial(
        pl.kernel,
        mesh=mesh,
        out_type=jax.ShapeDtypeStruct((n_total, 2 * dim), jnp.float32),
        scratch_types=[
            pltpu.VMEM((NIDX, CHUNK), jnp.int32),
            pltpu.VMEM((NROW, CHUNK, dim), jnp.float32),
            pltpu.SemaphoreType.DMA,
            pltpu.SemaphoreType.DMA,
            pltpu.SemaphoreType.DMA,
            pltpu.SemaphoreType.DMA,
            pltpu.SemaphoreType.DMA,
            pltpu.SemaphoreType.DMA,
            pltpu.SemaphoreType.DMA,
            pltpu.SemaphoreType.DMA,
            pltpu.SemaphoreType.DMA,
            pltpu.SemaphoreType.DMA,
            pltpu.SemaphoreType.DMA,
            pltpu.SemaphoreType.DMA,
        ],
        compiler_params=pltpu.CompilerParams(use_tc_tiling_on_sc=False),
    )
    def gather_kernel(idx_hbm, table_hbm, out_hbm, idx_v, rows_v,
                      isem0, isem1, isem2, isem3,
                      gsem0, gsem1, gsem2, gsem3,
                      osem0, osem1, osem2, osem3):
        isems = (isem0, isem1, isem2, isem3)
        gsems = (gsem0, gsem1, gsem2, gsem3)
        osems = (osem0, osem1, osem2, osem3)
        wid = lax.axis_index("s") * info.num_cores + lax.axis_index("c")
        base = wid * per_w

        def idx_load_start(c, j):
            pltpu.async_copy(
                idx_hbm.at[pl.ds(base + c * CHUNK, CHUNK)], idx_v.at[j], isems[j])

        def idx_load_wait(j):
            pltpu.make_async_copy(
                idx_hbm.at[pl.ds(base, CHUNK)], idx_v.at[j], isems[j]).wait()

        def gather_wait(b):
            pltpu.make_async_copy(
                table_hbm.at[idx_v.at[b]], rows_v.at[b], gsems[b]).wait()

        def write_start(c, b):
            pltpu.async_copy(
                rows_v.at[b],
                out_hbm.at[pl.ds(base + c * CHUNK, CHUNK), pl.ds(0, dim)],
                osems[b])

        def out_write_wait(b):
            pltpu.make_async_copy(
                rows_v.at[b],
                out_hbm.at[pl.ds(base, CHUNK), pl.ds(0, dim)], osems[b]).wait()

        def visit(c, k, first=False, prefetch=True):
            b = k % NROW
            if not first:
                out_write_wait(b)
            idx_load_wait(b)
            pltpu.async_copy(table_hbm.at[idx_v.at[b]], rows_v.at[b], gsems[b])
            if prefetch:
                idx_load_start(c + 2, (k + 2) % NIDX)
            kp = (k - 1) % NROW
            if not (first and k == 0):
                gather_wait(kp)
                write_start(c - 1, kp)

        n_quads = n_chunks // NIDX

        idx_load_start(0, 0)
        idx_load_start(1, 1)
        for k in range(NIDX):
            visit(k, k, first=True)

        def quad(o, carry):
            cb = o * NIDX
            for k in range(NIDX):
                visit(cb + k, k)
            return carry

        lax.fori_loop(1, n_quads - 1, quad, 0)

        cb = (n_quads - 1) * NIDX
        for k in range(NIDX):
            visit(cb + k, k, prefetch=(k < 2))

        last = n_chunks - 1
        bl = last % NROW
        gather_wait(bl)
        write_start(last, bl)
        for b in range(NROW):
            out_write_wait(b)

    return gather_kernel


def kernel(x, table):
    b, l = x.shape
    n = b * l
    flat = x.reshape(n).astype(jnp.int32)
    out_pad = _make_gather(n, DIM)(flat, table)
    return out_pad[:, :DIM].reshape(b, l, DIM)
